# Initial kernel scaffold; baseline (speedup 1.0000x reference)
import jax
import jax.numpy as jnp
from jax.experimental import pallas as pl


def kernel(tile37, dora_tile34, emb_t34, emb_suit, emb_rank, emb_honor, emb_red, emb_cls, emb_dora, proj_w, proj_b, ln_g, ln_b):
    raise NotImplementedError("write your pallas kernel here")


# speedup vs baseline: 21.8466x; 21.8466x over previous
import functools

import jax
import jax.numpy as jnp
from jax import lax
from jax.experimental import pallas as pl
from jax.experimental.pallas import tpu as pltpu
from jax.experimental.pallas import tpu_sc as plsc

OUT_DIM = 64
NROWS = 128
PAIR_ROWS = 114 * 128
PAIR_BLK = PAIR_ROWS // 8


def _tile_attrs(t):
    red = (t == 0) | (t == 10) | (t == 20)
    tm1 = t - 1 - jnp.where(t > 10, 1, 0) - jnp.where(t > 20, 1, 0)
    t34 = jnp.where(t == 0, 4,
          jnp.where(t == 10, 13,
          jnp.where(t == 20, 22,
          jnp.where(t >= 37, 34, tm1))))
    return red, t34


def _canonical_table(t34r, sur, rkr, hkr, redr, clsr, dorr, pwr, pbr, gr, br):
    i = lax.broadcasted_iota(jnp.int32, (NROWS, 1), 0)
    t = jnp.minimum(i // 3, 37)
    d = i % 3
    red, t34 = _tile_attrs(t)
    su = jnp.where(t34 == 34, 4,
         jnp.where(t34 <= 8, 0,
         jnp.where(t34 <= 17, 1,
         jnp.where(t34 <= 26, 2, 3))))
    rk26 = t34 - su * 9
    rk = jnp.where(t34 <= 26, rk26, jnp.where(t34 <= 33, 9, 10))
    hk = jnp.where(t34 <= 26, 7, jnp.where(t34 <= 33, t34 - 27, 8))
    rf = jnp.where(t >= 37, 2, jnp.where(red, 1, 0))
    tc = jnp.where(t34 <= 26, jnp.where((rk26 == 0) | (rk26 == 8), 1, 0),
         jnp.where(t34 <= 30, 2, jnp.where(t34 <= 33, 3, 4)))

    def sel(attr, emb_ref):
        vocab = emb_ref.shape[0]
        col = lax.broadcasted_iota(jnp.int32, (NROWS, vocab), 1)
        onehot = (col == attr).astype(jnp.float32)
        return jnp.dot(onehot, emb_ref[...], preferred_element_type=jnp.float32)

    parts = jnp.concatenate([
        sel(t34, t34r), sel(su, sur), sel(rk, rkr), sel(hk, hkr),
        sel(rf, redr), sel(tc, clsr), sel(d, dorr)], axis=1)
    h = jnp.dot(parts, pwr[...], preferred_element_type=jnp.float32) + pbr[...]
    mu = jnp.mean(h, axis=1, keepdims=True)
    xm = h - mu
    var = jnp.mean(xm * xm, axis=1, keepdims=True)
    return xm * lax.rsqrt(var + 1e-5) * gr[...] + br[...]


def _pair_table_body(t34r, sur, rkr, hkr, redr, clsr, dorr, pwr, pbr, gr, br,
                     out_ref):
    table = _canonical_table(t34r, sur, rkr, hkr, redr, clsr, dorr,
                             pwr, pbr, gr, br)
    base = pl.program_id(0) * PAIR_BLK
    i = base + lax.broadcasted_iota(jnp.int32, (PAIR_BLK, 1), 0)
    a = lax.shift_right_logical(i, 7)
    b = jnp.bitwise_and(i, 127)
    col = lax.broadcasted_iota(jnp.int32, (PAIR_BLK, NROWS), 1)
    left = jnp.dot((col == a).astype(jnp.float32), table,
                   preferred_element_type=jnp.float32)
    right = jnp.dot((col == b).astype(jnp.float32), table,
                    preferred_element_type=jnp.float32)
    out_ref[...] = jnp.concatenate([left, right], axis=1)


def _build_pair_table(emb_t34, emb_suit, emb_rank, emb_honor, emb_red,
                      emb_cls, emb_dora, proj_w, proj_b, ln_g, ln_b):
    full = lambda shape: pl.BlockSpec(shape, lambda i: (0, 0))
    return pl.pallas_call(
        _pair_table_body,
        grid=(8,),
        in_specs=[full(emb_t34.shape), full(emb_suit.shape),
                  full(emb_rank.shape), full(emb_honor.shape),
                  full(emb_red.shape), full(emb_cls.shape),
                  full(emb_dora.shape), full(proj_w.shape),
                  full((1, OUT_DIM)), full((1, OUT_DIM)), full((1, OUT_DIM))],
        out_specs=pl.BlockSpec((PAIR_BLK, 2 * OUT_DIM), lambda i: (i, 0)),
        out_shape=jax.ShapeDtypeStruct((PAIR_ROWS, 2 * OUT_DIM), jnp.float32),
    )(emb_t34, emb_suit, emb_rank, emb_honor, emb_red, emb_cls, emb_dora,
      proj_w, proj_b.reshape(1, OUT_DIM), ln_g.reshape(1, OUT_DIM),
      ln_b.reshape(1, OUT_DIM))


def _sc_embed(t37e, t37o, dora_flat, table2, B, L):
    P = L // 2
    info = plsc.get_sparse_core_info()
    nc, ns = info.num_cores, info.num_subcores
    nw = nc * ns
    rpw = B // nw
    unit = 2 * P
    nchunk = (unit + 15) // 16
    lp = nchunk * 16
    n_a = min(128, lp)
    n_b = lp - n_a
    mesh = plsc.VectorSubcoreMesh(core_axis_name="c", subcore_axis_name="s")

    @functools.partial(
        pl.kernel,
        out_type=jax.ShapeDtypeStruct((B * P, 2 * OUT_DIM), jnp.float32),
        mesh=mesh,
        scratch_types=[
            pltpu.VMEM((rpw * 4,), jnp.int32),
            pltpu.VMEM((4 * P + 16,), jnp.int32),
            pltpu.VMEM((4 * P + 16,), jnp.int32),
            pltpu.VMEM((n_a,), jnp.int32),
            pltpu.VMEM((n_b,), jnp.int32),
            pltpu.VMEM((lp, 2 * OUT_DIM), jnp.float32),
            pltpu.SemaphoreType.DMA,
            pltpu.SemaphoreType.DMA,
        ],
    )
    def k(t37e_hbm, t37o_hbm, dora_hbm, table_hbm, out_hbm,
          dora_v, ev, ov, idx_a, idx_b, rows_v, sem_a, sem_b):
        wid = lax.axis_index("s") * nc + lax.axis_index("c")
        base = wid * rpw
        pltpu.sync_copy(dora_hbm.at[pl.ds(base * 4, rpw * 4)], dora_v)

        def member(t34, dsp):
            return ((t34 == dsp[0]) | (t34 == dsp[1]) |
                    (t34 == dsp[2]) | (t34 == dsp[3]))

        def pair_idx(t, mem):
            red, _ = _tile_attrs(t)
            return jnp.where(t == 37, 2, jnp.where(red | mem, 0, 1)) + t * 3

        def group_body(g, carry):
            dchunk = dora_v[pl.ds(g * 16, 16)]
            pltpu.sync_copy(t37e_hbm.at[pl.ds((base + 4 * g) * P, 4 * P)],
                            ev.at[pl.ds(0, 4 * P)])
            pltpu.sync_copy(t37o_hbm.at[pl.ds((base + 4 * g) * P, 4 * P)],
                            ov.at[pl.ds(0, 4 * P)])
            for q in range(2):
                dspa = [jnp.broadcast_to(dchunk[8 * q + j], (16,))
                        for j in range(4)]
                dspb = [jnp.broadcast_to(dchunk[8 * q + 4 + j], (16,))
                        for j in range(4)]
                for c in range(nchunk):
                    off = q * unit + c * 16
                    te = jnp.clip(ev[pl.ds(off, 16)], 0, 37)
                    to = jnp.clip(ov[pl.ds(off, 16)], 0, 37)
                    _, t34e = _tile_attrs(te)
                    _, t34o = _tile_attrs(to)
                    if (c + 1) * 16 <= P:
                        meme, memo = member(t34e, dspa), member(t34o, dspa)
                    elif c * 16 >= P:
                        meme, memo = member(t34e, dspb), member(t34o, dspb)
                    else:
                        pos = c * 16 + lax.broadcasted_iota(jnp.int32, (16,), 0)
                        ina = pos < P
                        meme = jnp.where(ina, member(t34e, dspa),
                                         member(t34e, dspb))
                        memo = jnp.where(ina, member(t34o, dspa),
                                         member(t34o, dspb))
                    idx = pair_idx(te, meme) * 128 + pair_idx(to, memo)
                    if c * 16 < n_a:
                        idx_a[pl.ds(c * 16, 16)] = idx
                    else:
                        idx_b[pl.ds(c * 16 - n_a, 16)] = idx
                cp_a = pltpu.async_copy(table_hbm.at[idx_a],
                                        rows_v.at[pl.ds(0, n_a)], sem_a)
                cp_b = pltpu.async_copy(table_hbm.at[idx_b],
                                        rows_v.at[pl.ds(n_a, n_b)], sem_b)
                cp_a.wait()
                cp_b.wait()
                pltpu.sync_copy(
                    rows_v.at[pl.ds(0, unit)],
                    out_hbm.at[pl.ds((base + 4 * g + 2 * q) * P, unit)])
            return carry

        lax.fori_loop(0, rpw // 4, group_body, 0)

    return k(t37e, t37o, dora_flat, table2)


def kernel(tile37, dora_tile34, emb_t34, emb_suit, emb_rank, emb_honor,
           emb_red, emb_cls, emb_dora, proj_w, proj_b, ln_g, ln_b):
    B, L = tile37.shape
    table2 = _build_pair_table(emb_t34, emb_suit, emb_rank, emb_honor,
                               emb_red, emb_cls, emb_dora, proj_w, proj_b,
                               ln_g, ln_b)
    t37p = tile37.reshape(B, L // 2, 2)
    t37e = t37p[:, :, 0].reshape(-1)
    t37o = t37p[:, :, 1].reshape(-1)
    out = _sc_embed(t37e, t37o, dora_tile34.reshape(-1), table2, B, L)
    return out.reshape(B, L, OUT_DIM)


# speedup vs baseline: 25.9520x; 1.1879x over previous
import functools

import jax
import jax.numpy as jnp
from jax import lax
from jax.experimental import pallas as pl
from jax.experimental.pallas import tpu as pltpu
from jax.experimental.pallas import tpu_sc as plsc

OUT_DIM = 64
NROWS = 128
PAIR_ROWS = 114 * 128
PAIR_BLK = PAIR_ROWS // 8


def _tile_attrs(t):
    red = (t == 0) | (t == 10) | (t == 20)
    tm1 = t - 1 - jnp.where(t > 10, 1, 0) - jnp.where(t > 20, 1, 0)
    t34 = jnp.where(t == 0, 4,
          jnp.where(t == 10, 13,
          jnp.where(t == 20, 22,
          jnp.where(t >= 37, 34, tm1))))
    return red, t34


def _canonical_table(t34r, sur, rkr, hkr, redr, clsr, dorr, pwr, pbr, gr, br):
    i = lax.broadcasted_iota(jnp.int32, (NROWS, 1), 0)
    t = jnp.minimum(i // 3, 37)
    d = i % 3
    red, t34 = _tile_attrs(t)
    su = jnp.where(t34 == 34, 4,
         jnp.where(t34 <= 8, 0,
         jnp.where(t34 <= 17, 1,
         jnp.where(t34 <= 26, 2, 3))))
    rk26 = t34 - su * 9
    rk = jnp.where(t34 <= 26, rk26, jnp.where(t34 <= 33, 9, 10))
    hk = jnp.where(t34 <= 26, 7, jnp.where(t34 <= 33, t34 - 27, 8))
    rf = jnp.where(t >= 37, 2, jnp.where(red, 1, 0))
    tc = jnp.where(t34 <= 26, jnp.where((rk26 == 0) | (rk26 == 8), 1, 0),
         jnp.where(t34 <= 30, 2, jnp.where(t34 <= 33, 3, 4)))

    def sel(attr, emb_ref):
        vocab = emb_ref.shape[0]
        col = lax.broadcasted_iota(jnp.int32, (NROWS, vocab), 1)
        onehot = (col == attr).astype(jnp.float32)
        return jnp.dot(onehot, emb_ref[...], preferred_element_type=jnp.float32)

    parts = jnp.concatenate([
        sel(t34, t34r), sel(su, sur), sel(rk, rkr), sel(hk, hkr),
        sel(rf, redr), sel(tc, clsr), sel(d, dorr)], axis=1)
    h = jnp.dot(parts, pwr[...], preferred_element_type=jnp.float32) + pbr[...]
    mu = jnp.mean(h, axis=1, keepdims=True)
    xm = h - mu
    var = jnp.mean(xm * xm, axis=1, keepdims=True)
    return xm * lax.rsqrt(var + 1e-5) * gr[...] + br[...]


def _pair_table_body(t34r, sur, rkr, hkr, redr, clsr, dorr, pwr, pbr, gr, br,
                     out_ref):
    table = _canonical_table(t34r, sur, rkr, hkr, redr, clsr, dorr,
                             pwr, pbr, gr, br)
    base = pl.program_id(0) * PAIR_BLK
    i = base + lax.broadcasted_iota(jnp.int32, (PAIR_BLK, 1), 0)
    a = lax.shift_right_logical(i, 7)
    b = jnp.bitwise_and(i, 127)
    col = lax.broadcasted_iota(jnp.int32, (PAIR_BLK, NROWS), 1)
    left = jnp.dot((col == a).astype(jnp.float32), table,
                   preferred_element_type=jnp.float32)
    right = jnp.dot((col == b).astype(jnp.float32), table,
                    preferred_element_type=jnp.float32)
    out_ref[...] = jnp.concatenate([left, right], axis=1)


def _build_pair_table(emb_t34, emb_suit, emb_rank, emb_honor, emb_red,
                      emb_cls, emb_dora, proj_w, proj_b, ln_g, ln_b):
    full = lambda shape: pl.BlockSpec(shape, lambda i: (0, 0))
    return pl.pallas_call(
        _pair_table_body,
        grid=(8,),
        in_specs=[full(emb_t34.shape), full(emb_suit.shape),
                  full(emb_rank.shape), full(emb_honor.shape),
                  full(emb_red.shape), full(emb_cls.shape),
                  full(emb_dora.shape), full(proj_w.shape),
                  full((1, OUT_DIM)), full((1, OUT_DIM)), full((1, OUT_DIM))],
        out_specs=pl.BlockSpec((PAIR_BLK, 2 * OUT_DIM), lambda i: (i, 0)),
        out_shape=jax.ShapeDtypeStruct((PAIR_ROWS, 2 * OUT_DIM), jnp.float32),
    )(emb_t34, emb_suit, emb_rank, emb_honor, emb_red, emb_cls, emb_dora,
      proj_w, proj_b.reshape(1, OUT_DIM), ln_g.reshape(1, OUT_DIM),
      ln_b.reshape(1, OUT_DIM))


def _sc_embed(t37e, t37o, dora_flat, table2, B, L):
    P = L // 2
    info = plsc.get_sparse_core_info()
    nc, ns = info.num_cores, info.num_subcores
    nw = nc * ns
    rpw = B // nw
    unit = 2 * P
    nchunk = (unit + 15) // 16
    lp = nchunk * 16
    n_a = min(128, lp)
    n_b = lp - n_a
    mesh = plsc.VectorSubcoreMesh(core_axis_name="c", subcore_axis_name="s")

    @functools.partial(
        pl.kernel,
        out_type=jax.ShapeDtypeStruct((B * P, 2 * OUT_DIM), jnp.float32),
        mesh=mesh,
        scratch_types=[
            pltpu.VMEM((rpw * 4,), jnp.int32),
            pltpu.VMEM((rpw * P,), jnp.int32),
            pltpu.VMEM((rpw * P,), jnp.int32),
            pltpu.VMEM((n_a,), jnp.int32),
            pltpu.VMEM((n_b,), jnp.int32),
            pltpu.VMEM((n_a,), jnp.int32),
            pltpu.VMEM((n_b,), jnp.int32),
            pltpu.VMEM((lp, 2 * OUT_DIM), jnp.float32),
            pltpu.VMEM((lp, 2 * OUT_DIM), jnp.float32),
            pltpu.SemaphoreType.DMA,
            pltpu.SemaphoreType.DMA,
            pltpu.SemaphoreType.DMA,
            pltpu.SemaphoreType.DMA,
            pltpu.SemaphoreType.DMA,
            pltpu.SemaphoreType.DMA,
        ],
    )
    def k(t37e_hbm, t37o_hbm, dora_hbm, table_hbm, out_hbm,
          dora_v, ev, ov, ia0, ib0, ia1, ib1, rows0, rows1,
          ga0, gb0, ga1, gb1, ss0, ss1):
        wid = lax.axis_index("s") * nc + lax.axis_index("c")
        base = wid * rpw
        pltpu.sync_copy(dora_hbm.at[pl.ds(base * 4, rpw * 4)], dora_v)
        pltpu.sync_copy(t37e_hbm.at[pl.ds(base * P, rpw * P)], ev)
        pltpu.sync_copy(t37o_hbm.at[pl.ds(base * P, rpw * P)], ov)
        bufs = [(ia0, ib0, rows0, ga0, gb0, ss0),
                (ia1, ib1, rows1, ga1, gb1, ss1)]

        def member(t34, dsp):
            return ((t34 == dsp[0]) | (t34 == dsp[1]) |
                    (t34 == dsp[2]) | (t34 == dsp[3]))

        def pair_idx(t, mem):
            red, _ = _tile_attrs(t)
            return jnp.where(t == 37, 2, jnp.where(red | mem, 0, 1)) + t * 3

        def group_body(g, carry):
            dchunk = dora_v[pl.ds(g * 16, 16)]
            for q in range(2):
                idx_a, idx_b, rows_v, sem_a, sem_b, sem_s = bufs[q]
                u = 2 * g + q
                dspa = [jnp.broadcast_to(dchunk[8 * q + j], (16,))
                        for j in range(4)]
                dspb = [jnp.broadcast_to(dchunk[8 * q + 4 + j], (16,))
                        for j in range(4)]
                for c in range(nchunk):
                    off = u * unit + c * 16
                    te = jnp.clip(ev[pl.ds(off, 16)], 0, 37)
                    to = jnp.clip(ov[pl.ds(off, 16)], 0, 37)
                    _, t34e = _tile_attrs(te)
                    _, t34o = _tile_attrs(to)
                    if (c + 1) * 16 <= P:
                        meme, memo = member(t34e, dspa), member(t34o, dspa)
                    elif c * 16 >= P:
                        meme, memo = member(t34e, dspb), member(t34o, dspb)
                    else:
                        pos = c * 16 + lax.broadcasted_iota(jnp.int32, (16,), 0)
                        ina = pos < P
                        meme = jnp.where(ina, member(t34e, dspa),
                                         member(t34e, dspb))
                        memo = jnp.where(ina, member(t34o, dspa),
                                         member(t34o, dspb))
                    idx = pair_idx(te, meme) * 128 + pair_idx(to, memo)
                    if c * 16 < n_a:
                        idx_a[pl.ds(c * 16, 16)] = idx
                    else:
                        idx_b[pl.ds(c * 16 - n_a, 16)] = idx

                @pl.when(g > 0)
                def _():
                    pltpu.make_async_copy(
                        rows_v.at[pl.ds(0, unit)],
                        out_hbm.at[pl.ds((base + 2 * (u - 2)) * P, unit)],
                        sem_s).wait()

                cp_a = pltpu.async_copy(table_hbm.at[idx_a],
                                        rows_v.at[pl.ds(0, n_a)], sem_a)
                cp_b = pltpu.async_copy(table_hbm.at[idx_b],
                                        rows_v.at[pl.ds(n_a, n_b)], sem_b)
                cp_a.wait()
                cp_b.wait()
                pltpu.async_copy(rows_v.at[pl.ds(0, unit)],
                                 out_hbm.at[pl.ds((base + 2 * u) * P, unit)],
                                 sem_s)
            return carry

        nunits = rpw // 2
        lax.fori_loop(0, nunits // 2, group_body, 0)
        for q in range(2):
            _, _, rows_v, _, _, sem_s = bufs[q]
            u_last = nunits - 2 + q
            pltpu.make_async_copy(
                rows_v.at[pl.ds(0, unit)],
                out_hbm.at[pl.ds((base + 2 * u_last) * P, unit)],
                sem_s).wait()

    return k(t37e, t37o, dora_flat, table2)


def kernel(tile37, dora_tile34, emb_t34, emb_suit, emb_rank, emb_honor,
           emb_red, emb_cls, emb_dora, proj_w, proj_b, ln_g, ln_b):
    B, L = tile37.shape
    table2 = _build_pair_table(emb_t34, emb_suit, emb_rank, emb_honor,
                               emb_red, emb_cls, emb_dora, proj_w, proj_b,
                               ln_g, ln_b)
    t37p = tile37.reshape(B, L // 2, 2)
    t37e = t37p[:, :, 0].reshape(-1)
    t37o = t37p[:, :, 1].reshape(-1)
    out = _sc_embed(t37e, t37o, dora_tile34.reshape(-1), table2, B, L)
    return out.reshape(B, L, OUT_DIM)


# speedup vs baseline: 26.0794x; 1.0049x over previous
import functools

import jax
import jax.numpy as jnp
from jax import lax
from jax.experimental import pallas as pl
from jax.experimental.pallas import tpu as pltpu
from jax.experimental.pallas import tpu_sc as plsc

OUT_DIM = 64
NROWS = 128
PAIR_ROWS = 114 * 128
PAIR_BLK = PAIR_ROWS // 8
IDX_BLK = 512


def _tile_attrs(t):
    red = (t == 0) | (t == 10) | (t == 20)
    tm1 = t - 1 - jnp.where(t > 10, 1, 0) - jnp.where(t > 20, 1, 0)
    t34 = jnp.where(t == 0, 4,
          jnp.where(t == 10, 13,
          jnp.where(t == 20, 22,
          jnp.where(t >= 37, 34, tm1))))
    return red, t34


def _canonical_table(t34r, sur, rkr, hkr, redr, clsr, dorr, pwr, pbr, gr, br):
    i = lax.broadcasted_iota(jnp.int32, (NROWS, 1), 0)
    t = jnp.minimum(i // 3, 37)
    d = i % 3
    red, t34 = _tile_attrs(t)
    su = jnp.where(t34 == 34, 4,
         jnp.where(t34 <= 8, 0,
         jnp.where(t34 <= 17, 1,
         jnp.where(t34 <= 26, 2, 3))))
    rk26 = t34 - su * 9
    rk = jnp.where(t34 <= 26, rk26, jnp.where(t34 <= 33, 9, 10))
    hk = jnp.where(t34 <= 26, 7, jnp.where(t34 <= 33, t34 - 27, 8))
    rf = jnp.where(t >= 37, 2, jnp.where(red, 1, 0))
    tc = jnp.where(t34 <= 26, jnp.where((rk26 == 0) | (rk26 == 8), 1, 0),
         jnp.where(t34 <= 30, 2, jnp.where(t34 <= 33, 3, 4)))

    def sel(attr, emb_ref):
        vocab = emb_ref.shape[0]
        col = lax.broadcasted_iota(jnp.int32, (NROWS, vocab), 1)
        onehot = (col == attr).astype(jnp.float32)
        return jnp.dot(onehot, emb_ref[...], preferred_element_type=jnp.float32)

    parts = jnp.concatenate([
        sel(t34, t34r), sel(su, sur), sel(rk, rkr), sel(hk, hkr),
        sel(rf, redr), sel(tc, clsr), sel(d, dorr)], axis=1)
    h = jnp.dot(parts, pwr[...], preferred_element_type=jnp.float32) + pbr[...]
    mu = jnp.mean(h, axis=1, keepdims=True)
    xm = h - mu
    var = jnp.mean(xm * xm, axis=1, keepdims=True)
    return xm * lax.rsqrt(var + 1e-5) * gr[...] + br[...]


def _pair_table_body(t34r, sur, rkr, hkr, redr, clsr, dorr, pwr, pbr, gr, br,
                     out_ref):
    table = _canonical_table(t34r, sur, rkr, hkr, redr, clsr, dorr,
                             pwr, pbr, gr, br)
    base = pl.program_id(0) * PAIR_BLK
    i = base + lax.broadcasted_iota(jnp.int32, (PAIR_BLK, 1), 0)
    a = lax.shift_right_logical(i, 7)
    b = jnp.bitwise_and(i, 127)
    col = lax.broadcasted_iota(jnp.int32, (PAIR_BLK, NROWS), 1)
    left = jnp.dot((col == a).astype(jnp.float32), table,
                   preferred_element_type=jnp.float32)
    right = jnp.dot((col == b).astype(jnp.float32), table,
                    preferred_element_type=jnp.float32)
    out_ref[...] = jnp.concatenate([left, right], axis=1)


def _build_pair_table(emb_t34, emb_suit, emb_rank, emb_honor, emb_red,
                      emb_cls, emb_dora, proj_w, proj_b, ln_g, ln_b):
    full = lambda shape: pl.BlockSpec(shape, lambda i: (0, 0))
    return pl.pallas_call(
        _pair_table_body,
        grid=(8,),
        in_specs=[full(emb_t34.shape), full(emb_suit.shape),
                  full(emb_rank.shape), full(emb_honor.shape),
                  full(emb_red.shape), full(emb_cls.shape),
                  full(emb_dora.shape), full(proj_w.shape),
                  full((1, OUT_DIM)), full((1, OUT_DIM)), full((1, OUT_DIM))],
        out_specs=pl.BlockSpec((PAIR_BLK, 2 * OUT_DIM), lambda i: (i, 0)),
        out_shape=jax.ShapeDtypeStruct((PAIR_ROWS, 2 * OUT_DIM), jnp.float32),
    )(emb_t34, emb_suit, emb_rank, emb_honor, emb_red, emb_cls, emb_dora,
      proj_w, proj_b.reshape(1, OUT_DIM), ln_g.reshape(1, OUT_DIM),
      ln_b.reshape(1, OUT_DIM))


def _pair_index_body(L, t37r, dorar, out_ref):
    t = t37r[...]
    red, t34 = _tile_attrs(t)
    mem = (t34 == dorar[:, 0:1]) | (t34 == dorar[:, 1:2]) | \
          (t34 == dorar[:, 2:3]) | (t34 == dorar[:, 3:4])
    idx = jnp.where(t == 37, 2, jnp.where(red | mem, 0, 1)) + t * 3
    idxf = idx.astype(jnp.float32)
    P = L // 2
    row = lax.broadcasted_iota(jnp.int32, (L, P), 0)
    colp = lax.broadcasted_iota(jnp.int32, (L, P), 1)
    sel_e = (row == 2 * colp).astype(jnp.float32)
    sel_o = (row == 2 * colp + 1).astype(jnp.float32)
    e = jnp.dot(idxf, sel_e, preferred_element_type=jnp.float32)
    o = jnp.dot(idxf, sel_o, preferred_element_type=jnp.float32)
    out_ref[...] = (e * 128.0 + o).astype(jnp.int32)


def _pair_indices(tile37, dora_tile34, B, L):
    P = L // 2
    return pl.pallas_call(
        functools.partial(_pair_index_body, L),
        grid=(B // IDX_BLK,),
        in_specs=[pl.BlockSpec((IDX_BLK, L), lambda i: (i, 0)),
                  pl.BlockSpec((IDX_BLK, 4), lambda i: (i, 0))],
        out_specs=pl.BlockSpec((IDX_BLK, P), lambda i: (i, 0)),
        out_shape=jax.ShapeDtypeStruct((B, P), jnp.int32),
    )(tile37, dora_tile34).reshape(B * P)


def _sc_embed(pidx, table2, B, L):
    P = L // 2
    info = plsc.get_sparse_core_info()
    nc, ns = info.num_cores, info.num_subcores
    nw = nc * ns
    rpw = B // nw
    unit = 2 * P
    lp = ((unit + 15) // 16) * 16
    n_a = min(128, unit)
    n_b = unit - n_a
    nbuf = 4
    nunits = rpw // 2
    mesh = plsc.VectorSubcoreMesh(core_axis_name="c", subcore_axis_name="s")

    @functools.partial(
        pl.kernel,
        out_type=jax.ShapeDtypeStruct((B * P, 2 * OUT_DIM), jnp.float32),
        mesh=mesh,
        scratch_types=[
            pltpu.VMEM((rpw * P,), jnp.int32),
            *[pltpu.VMEM((unit, 2 * OUT_DIM), jnp.float32)
              for _ in range(nbuf)],
            *[pltpu.SemaphoreType.DMA for _ in range(3 * nbuf)],
        ],
    )
    def k(pidx_hbm, table_hbm, out_hbm, idx_v,
          rows0, rows1, rows2, rows3,
          ga0, gb0, ss0, ga1, gb1, ss1, ga2, gb2, ss2, ga3, gb3, ss3):
        wid = lax.axis_index("s") * nc + lax.axis_index("c")
        base = wid * rpw
        pbase = base * P
        pltpu.sync_copy(pidx_hbm.at[pl.ds(pbase, rpw * P)], idx_v)
        bufs = [(rows0, ga0, gb0, ss0), (rows1, ga1, gb1, ss1),
                (rows2, ga2, gb2, ss2), (rows3, ga3, gb3, ss3)]

        def gather_copies(u, slot):
            rows_v, sem_a, sem_b, _ = bufs[slot]
            return (
                pltpu.make_async_copy(
                    table_hbm.at[idx_v.at[pl.ds(u * unit, n_a)]],
                    rows_v.at[pl.ds(0, n_a)], sem_a),
                pltpu.make_async_copy(
                    table_hbm.at[idx_v.at[pl.ds(u * unit + n_a, n_b)]],
                    rows_v.at[pl.ds(n_a, n_b)], sem_b),
            )

        def scatter_copy(u, slot):
            rows_v, _, _, sem_s = bufs[slot]
            return pltpu.make_async_copy(
                rows_v.at[pl.ds(0, unit)],
                out_hbm.at[pl.ds(pbase + u * unit, unit)], sem_s)

        def start_gather(u, slot):
            for cp in gather_copies(u, slot):
                cp.start()

        def wait_gather(u, slot):
            for cp in gather_copies(u, slot):
                cp.wait()

        for j in range(nbuf):
            start_gather(j, j)

        def step_body(m, carry):
            for j in range(nbuf):
                u = nbuf * m + j
                slot_n = (j + 1) % nbuf
                wait_gather(u, j)
                scatter_copy(u, j).start()

                @pl.when((u >= nbuf - 1) & (u <= nunits - 2))
                def _():
                    scatter_copy(u + 1 - nbuf, slot_n).wait()
                    start_gather(u + 1, slot_n)
            return carry

        lax.fori_loop(0, nunits // nbuf, step_body, 0)
        for j in range(nbuf):
            u_last = nunits - nbuf + j
            scatter_copy(u_last, (u_last % nbuf)).wait()

    return k(pidx, table2)


def kernel(tile37, dora_tile34, emb_t34, emb_suit, emb_rank, emb_honor,
           emb_red, emb_cls, emb_dora, proj_w, proj_b, ln_g, ln_b):
    B, L = tile37.shape
    table2 = _build_pair_table(emb_t34, emb_suit, emb_rank, emb_honor,
                               emb_red, emb_cls, emb_dora, proj_w, proj_b,
                               ln_g, ln_b)
    pidx = _pair_indices(tile37, dora_tile34, B, L)
    out = _sc_embed(pidx, table2, B, L)
    return out.reshape(B, L, OUT_DIM)


# speedup vs baseline: 78.1614x; 2.9971x over previous
import functools

import jax
import jax.numpy as jnp
from jax import lax
from jax.experimental import pallas as pl
from jax.experimental.pallas import tpu as pltpu
from jax.experimental.pallas import tpu_sc as plsc

OUT_DIM = 64
NROWS = 128
PAIR_ROWS = 114 * 128
PAIR_BLK = PAIR_ROWS // 8
IDX_BLK = 512


def _tile_attrs(t):
    red = (t == 0) | (t == 10) | (t == 20)
    tm1 = t - 1 - jnp.where(t > 10, 1, 0) - jnp.where(t > 20, 1, 0)
    t34 = jnp.where(t == 0, 4,
          jnp.where(t == 10, 13,
          jnp.where(t == 20, 22,
          jnp.where(t >= 37, 34, tm1))))
    return red, t34


def _canonical_table(t34r, sur, rkr, hkr, redr, clsr, dorr, pwr, pbr, gr, br):
    i = lax.broadcasted_iota(jnp.int32, (NROWS, 1), 0)
    t = jnp.minimum(i // 3, 37)
    d = i % 3
    red, t34 = _tile_attrs(t)
    su = jnp.where(t34 == 34, 4,
         jnp.where(t34 <= 8, 0,
         jnp.where(t34 <= 17, 1,
         jnp.where(t34 <= 26, 2, 3))))
    rk26 = t34 - su * 9
    rk = jnp.where(t34 <= 26, rk26, jnp.where(t34 <= 33, 9, 10))
    hk = jnp.where(t34 <= 26, 7, jnp.where(t34 <= 33, t34 - 27, 8))
    rf = jnp.where(t >= 37, 2, jnp.where(red, 1, 0))
    tc = jnp.where(t34 <= 26, jnp.where((rk26 == 0) | (rk26 == 8), 1, 0),
         jnp.where(t34 <= 30, 2, jnp.where(t34 <= 33, 3, 4)))

    def sel(attr, emb_ref):
        vocab = emb_ref.shape[0]
        col = lax.broadcasted_iota(jnp.int32, (NROWS, vocab), 1)
        onehot = (col == attr).astype(jnp.float32)
        return jnp.dot(onehot, emb_ref[...], preferred_element_type=jnp.float32)

    parts = jnp.concatenate([
        sel(t34, t34r), sel(su, sur), sel(rk, rkr), sel(hk, hkr),
        sel(rf, redr), sel(tc, clsr), sel(d, dorr)], axis=1)
    h = jnp.dot(parts, pwr[...], preferred_element_type=jnp.float32) + pbr[...]
    mu = jnp.mean(h, axis=1, keepdims=True)
    xm = h - mu
    var = jnp.mean(xm * xm, axis=1, keepdims=True)
    return xm * lax.rsqrt(var + 1e-5) * gr[...] + br[...]


def _pair_table_body(t34r, sur, rkr, hkr, redr, clsr, dorr, pwr, pbr, gr, br,
                     out_ref):
    table = _canonical_table(t34r, sur, rkr, hkr, redr, clsr, dorr,
                             pwr, pbr, gr, br)
    base = pl.program_id(0) * PAIR_BLK
    i = base + lax.broadcasted_iota(jnp.int32, (PAIR_BLK, 1), 0)
    a = lax.shift_right_logical(i, 7)
    b = jnp.bitwise_and(i, 127)
    col = lax.broadcasted_iota(jnp.int32, (PAIR_BLK, NROWS), 1)
    left = jnp.dot((col == a).astype(jnp.float32), table,
                   preferred_element_type=jnp.float32)
    right = jnp.dot((col == b).astype(jnp.float32), table,
                    preferred_element_type=jnp.float32)
    out_ref[...] = jnp.concatenate([left, right], axis=1)


def _build_pair_table(emb_t34, emb_suit, emb_rank, emb_honor, emb_red,
                      emb_cls, emb_dora, proj_w, proj_b, ln_g, ln_b):
    full = lambda shape: pl.BlockSpec(shape, lambda i: (0, 0))
    return pl.pallas_call(
        _pair_table_body,
        grid=(8,),
        in_specs=[full(emb_t34.shape), full(emb_suit.shape),
                  full(emb_rank.shape), full(emb_honor.shape),
                  full(emb_red.shape), full(emb_cls.shape),
                  full(emb_dora.shape), full(proj_w.shape),
                  full((1, OUT_DIM)), full((1, OUT_DIM)), full((1, OUT_DIM))],
        out_specs=pl.BlockSpec((PAIR_BLK, 2 * OUT_DIM), lambda i: (i, 0)),
        out_shape=jax.ShapeDtypeStruct((PAIR_ROWS, 2 * OUT_DIM), jnp.float32),
    )(emb_t34, emb_suit, emb_rank, emb_honor, emb_red, emb_cls, emb_dora,
      proj_w, proj_b.reshape(1, OUT_DIM), ln_g.reshape(1, OUT_DIM),
      ln_b.reshape(1, OUT_DIM))


def _pair_index_body(L, t37r, dorar, out_ref):
    t = t37r[...]
    red, t34 = _tile_attrs(t)
    mem = (t34 == dorar[:, 0:1]) | (t34 == dorar[:, 1:2]) | \
          (t34 == dorar[:, 2:3]) | (t34 == dorar[:, 3:4])
    idx = jnp.where(t == 37, 2, jnp.where(red | mem, 0, 1)) + t * 3
    idxf = idx.astype(jnp.float32)
    P = L // 2
    row = lax.broadcasted_iota(jnp.int32, (L, P), 0)
    colp = lax.broadcasted_iota(jnp.int32, (L, P), 1)
    sel_e = (row == 2 * colp).astype(jnp.float32)
    sel_o = (row == 2 * colp + 1).astype(jnp.float32)
    e = jnp.dot(idxf, sel_e, preferred_element_type=jnp.float32)
    o = jnp.dot(idxf, sel_o, preferred_element_type=jnp.float32)
    out_ref[...] = (e * 128.0 + o).astype(jnp.int32)


def _pair_indices(tile37, dora_tile34, B, L):
    P = L // 2
    return pl.pallas_call(
        functools.partial(_pair_index_body, L),
        grid=(B // IDX_BLK,),
        in_specs=[pl.BlockSpec((IDX_BLK, L), lambda i: (i, 0)),
                  pl.BlockSpec((IDX_BLK, 4), lambda i: (i, 0))],
        out_specs=pl.BlockSpec((IDX_BLK, P), lambda i: (i, 0)),
        out_shape=jax.ShapeDtypeStruct((B, P), jnp.int32),
    )(tile37, dora_tile34).reshape(B * P)


def _sc_embed(pidx, table2, B, L):
    P = L // 2
    info = plsc.get_sparse_core_info()
    nc, ns = info.num_cores, info.num_subcores
    nw = nc * ns
    rpw = B // nw
    unit = 2 * P
    lp = ((unit + 15) // 16) * 16
    n_a = min(128, unit)
    n_b = unit - n_a
    nbuf = 4
    nunits = rpw // 2
    mesh = plsc.VectorSubcoreMesh(core_axis_name="c", subcore_axis_name="s")

    @functools.partial(
        pl.kernel,
        out_type=jax.ShapeDtypeStruct((B * P, 2 * OUT_DIM), jnp.float32),
        mesh=mesh,
        scratch_types=[
            pltpu.VMEM((rpw * P,), jnp.int32),
            *[pltpu.VMEM((unit, 2 * OUT_DIM), jnp.float32)
              for _ in range(nbuf)],
            *[pltpu.SemaphoreType.DMA for _ in range(3 * nbuf)],
        ],
    )
    def k(pidx_hbm, table_hbm, out_hbm, idx_v,
          rows0, rows1, rows2, rows3,
          ga0, gb0, ss0, ga1, gb1, ss1, ga2, gb2, ss2, ga3, gb3, ss3):
        wid = lax.axis_index("s") * nc + lax.axis_index("c")
        base = wid * rpw
        pbase = base * P
        pltpu.sync_copy(pidx_hbm.at[pl.ds(pbase, rpw * P)], idx_v)
        bufs = [(rows0, ga0, gb0, ss0), (rows1, ga1, gb1, ss1),
                (rows2, ga2, gb2, ss2), (rows3, ga3, gb3, ss3)]

        def gather_copies(u, slot):
            rows_v, sem_a, sem_b, _ = bufs[slot]
            return (
                pltpu.make_async_copy(
                    table_hbm.at[idx_v.at[pl.ds(u * unit, n_a)]],
                    rows_v.at[pl.ds(0, n_a)], sem_a),
                pltpu.make_async_copy(
                    table_hbm.at[idx_v.at[pl.ds(u * unit + n_a, n_b)]],
                    rows_v.at[pl.ds(n_a, n_b)], sem_b),
            )

        def scatter_copy(u, slot):
            rows_v, _, _, sem_s = bufs[slot]
            return pltpu.make_async_copy(
                rows_v.at[pl.ds(0, unit)],
                out_hbm.at[pl.ds(pbase + u * unit, unit)], sem_s)

        def start_gather(u, slot):
            for cp in gather_copies(u, slot):
                cp.start()

        def wait_gather(u, slot):
            for cp in gather_copies(u, slot):
                cp.wait()

        for j in range(nbuf):
            start_gather(j, j)

        def step_body(m, carry):
            for j in range(nbuf):
                u = nbuf * m + j
                slot_n = (j + 1) % nbuf
                wait_gather(u, j)
                scatter_copy(u, j).start()

                @pl.when((u >= nbuf - 1) & (u <= nunits - 2))
                def _():
                    scatter_copy(u + 1 - nbuf, slot_n).wait()
                    start_gather(u + 1, slot_n)
            return carry

        lax.fori_loop(0, nunits // nbuf, step_body, 0)
        for j in range(nbuf):
            u_last = nunits - nbuf + j
            scatter_copy(u_last, (u_last % nbuf)).wait()

    return k(pidx, table2)


def kernel(tile37, dora_tile34, emb_t34, emb_suit, emb_rank, emb_honor,
           emb_red, emb_cls, emb_dora, proj_w, proj_b, ln_g, ln_b):
    B, L = tile37.shape
    table2 = _build_pair_table(emb_t34, emb_suit, emb_rank, emb_honor,
                               emb_red, emb_cls, emb_dora, proj_w, proj_b,
                               ln_g, ln_b)
    pidx = _pair_indices(tile37, dora_tile34, B, L)
    out = _sc_embed(pidx, table2, B, L)
    return out
